# baseline (speedup 1.0000x reference)
import functools

import jax
import jax.numpy as jnp
from jax import lax
from jax.experimental import pallas as pl
from jax.experimental.pallas import tpu as pltpu
from jax.experimental.pallas import tpu_sc as plsc

VOCAB = 100000
EMBED = 128
CTX = 200
HID = 64

_NC, _NS = 2, 16
_NW = _NC * _NS
PAD_B = 256
_BPW = PAD_B // _NW

TILE_V = 20000
NT = VOCAB // TILE_V


def _make_sc_gather():
    mesh = plsc.VectorSubcoreMesh(core_axis_name="c", subcore_axis_name="s")

    @functools.partial(
        pl.kernel,
        mesh=mesh,
        out_type=jax.ShapeDtypeStruct((PAD_B, EMBED), jnp.float32),
        scratch_types=[
            pltpu.VMEM((_BPW,), jnp.int32),
            pltpu.VMEM((_BPW, EMBED), jnp.float32),
            pltpu.SemaphoreType.DMA,
        ],
    )
    def sc_gather(idx_hbm, table_hbm, out_hbm, idx_v, rows_v, sem):
        wid = lax.axis_index("s") * _NC + lax.axis_index("c")
        base = wid * _BPW
        pltpu.sync_copy(idx_hbm.at[pl.ds(base, _BPW)], idx_v)
        pltpu.async_copy(table_hbm.at[idx_v], rows_v, sem).wait()
        pltpu.sync_copy(rows_v, out_hbm.at[pl.ds(base, _BPW)])

    return sc_gather


_sc_gather_cache = []


def _sc_gather(idx, table):
    if not _sc_gather_cache:
        _sc_gather_cache.append(_make_sc_gather())
    return _sc_gather_cache[0](idx, table)


def _mlp_body(e_ref, w1_ref, b1_ref, w2_ref, b2_ref, out_ref, h_ref):
    i = pl.program_id(0)

    @pl.when(i == 0)
    def _():
        h = lax.dot_general(
            e_ref[...], w1_ref[...], (((1,), (1,)), ((), ())),
            preferred_element_type=jnp.float32,
        )
        h_ref[...] = jnp.maximum(h + b1_ref[...], 0.0)

    o = lax.dot_general(
        h_ref[...], w2_ref[...], (((1,), (1,)), ((), ())),
        preferred_element_type=jnp.float32,
    ) + b2_ref[0]
    out_ref[pl.ds(i, 1), :] = o

    @pl.when(i == NT - 1)
    def _():
        x = out_ref[...]
        m = jnp.max(x)
        out_ref[...] = x - m - jnp.log(jnp.sum(jnp.exp(x - m)))


def _w2_body(b1_ref, w2_ref, b2_ref, out_ref, h_ref):
    i = pl.program_id(0)

    @pl.when(i == 0)
    def _():
        h_ref[...] = b1_ref[...]

    o = lax.dot_general(
        jnp.concatenate([h_ref[...], h_ref[...]], axis=1), w2_ref[...],
        (((1,), (1,)), ((), ())),
        preferred_element_type=jnp.float32,
    )
    out_ref[pl.ds(i, 1), :] = jnp.concatenate([o, o], axis=1) + b2_ref[0]

    @pl.when(i == NT - 1)
    def _():
        x = out_ref[...]
        m = jnp.max(x)
        out_ref[...] = x - m - jnp.log(jnp.sum(jnp.exp(x - m)))


def kernel(inputs, table, W1, b1, W2, b2):
    out = pl.pallas_call(
        _w2_body,
        grid=(NT,),
        in_specs=[
            pl.BlockSpec((1, HID), lambda i: (0, 0)),
            pl.BlockSpec((TILE_V // 2, 2 * HID), lambda i: (i, 0)),
            pl.BlockSpec((1, 1, TILE_V), lambda i: (i, 0, 0)),
        ],
        out_specs=pl.BlockSpec((NT, TILE_V), lambda i: (0, 0)),
        out_shape=jax.ShapeDtypeStruct((NT, TILE_V), jnp.float32),
        scratch_shapes=[pltpu.VMEM((1, HID), jnp.float32)],
    )(b1.reshape(1, HID), W2.reshape(VOCAB // 2, 2 * HID), b2.reshape(NT, 1, TILE_V))
    return out.reshape(1, VOCAB)


def _full_kernel(inputs, table, W1, b1, W2, b2):
    idx = jnp.zeros((PAD_B,), jnp.int32).at[:CTX].set(inputs)
    rows = _sc_gather(idx, table)
    e = rows[:CTX].reshape(1, CTX * EMBED)

    out = pl.pallas_call(
        _mlp_body,
        grid=(NT,),
        in_specs=[
            pl.BlockSpec((1, CTX * EMBED), lambda i: (0, 0)),
            pl.BlockSpec((HID, CTX * EMBED), lambda i: (0, 0)),
            pl.BlockSpec((1, HID), lambda i: (0, 0)),
            pl.BlockSpec((TILE_V, HID), lambda i: (i, 0)),
            pl.BlockSpec((1, 1, TILE_V), lambda i: (i, 0, 0)),
        ],
        out_specs=pl.BlockSpec((NT, TILE_V), lambda i: (0, 0)),
        out_shape=jax.ShapeDtypeStruct((NT, TILE_V), jnp.float32),
        scratch_shapes=[pltpu.VMEM((1, HID), jnp.float32)],
    )(e, W1, b1.reshape(1, HID), W2, b2.reshape(NT, 1, TILE_V))
    return out.reshape(1, VOCAB)


# speedup vs baseline: 2.4921x; 2.4921x over previous
import functools

import jax
import jax.numpy as jnp
from jax import lax
from jax.experimental import pallas as pl
from jax.experimental.pallas import tpu as pltpu
from jax.experimental.pallas import tpu_sc as plsc

VOCAB = 100000
EMBED = 128
CTX = 200
HID = 64

_NC, _NS = 2, 16
_BPW = 8
_NWORK = CTX // _BPW

TILE = 12800
NT = -(-VOCAB // TILE)


def _make_sc_gather():
    mesh = plsc.VectorSubcoreMesh(core_axis_name="c", subcore_axis_name="s")

    @functools.partial(
        pl.kernel,
        mesh=mesh,
        out_type=jax.ShapeDtypeStruct((CTX, EMBED), jnp.float32),
        scratch_types=[
            pltpu.VMEM((_BPW,), jnp.int32),
            pltpu.VMEM((_BPW, EMBED), jnp.float32),
            pltpu.SemaphoreType.DMA,
        ],
    )
    def sc_gather(idx_hbm, table_hbm, out_hbm, idx_v, rows_v, sem):
        wid = lax.axis_index("s") * _NC + lax.axis_index("c")

        @pl.when(wid < _NWORK)
        def _():
            base = wid * _BPW
            pltpu.sync_copy(idx_hbm.at[pl.ds(base, _BPW)], idx_v)
            pltpu.async_copy(table_hbm.at[idx_v], rows_v, sem).wait()
            pltpu.sync_copy(rows_v, out_hbm.at[pl.ds(base, _BPW)])

    return sc_gather


_sc_gather_cache = []


def _sc_gather(idx, table):
    if not _sc_gather_cache:
        _sc_gather_cache.append(_make_sc_gather())
    return _sc_gather_cache[0](idx, table)


def _mlp_body(e_ref, w1_ref, b1_ref, w2t_ref, b2_ref, out_ref,
              h_ref, o_scr, logz_ref):
    s = pl.program_id(0)

    @pl.when(s == 0)
    def _():
        h = lax.dot_general(
            e_ref[...], w1_ref[...], (((1,), (1,)), ((), ())),
            preferred_element_type=jnp.float32,
        )
        h_ref[...] = jnp.maximum(h + b1_ref[...], 0.0)

    @pl.when(s < NT)
    def _():
        o = lax.dot_general(
            h_ref[...], w2t_ref[...], (((1,), (0,)), ((), ())),
            preferred_element_type=jnp.float32,
        ) + b2_ref[...]
        o_scr[:, pl.ds(s * TILE, TILE)] = o

    @pl.when(s == NT - 1)
    def _():
        x = o_scr[...]
        col = lax.broadcasted_iota(jnp.int32, x.shape, 1)
        xm = jnp.where(col < VOCAB, x, -jnp.inf)
        m = jnp.max(xm)
        logz_ref[0, 0] = m + jnp.log(jnp.sum(jnp.exp(xm - m)))

    @pl.when(s >= NT)
    def _():
        j = s - NT
        out_ref[...] = o_scr[:, pl.ds(j * TILE, TILE)] - logz_ref[0, 0]


def kernel(inputs, table, W1, b1, W2, b2):
    rows = _sc_gather(inputs, table)
    e = rows.reshape(1, CTX * EMBED)
    w2t = W2.T

    out = pl.pallas_call(
        _mlp_body,
        grid=(2 * NT,),
        in_specs=[
            pl.BlockSpec((1, CTX * EMBED), lambda s: (0, 0)),
            pl.BlockSpec((HID, CTX * EMBED), lambda s: (0, 0)),
            pl.BlockSpec((1, HID), lambda s: (0, 0)),
            pl.BlockSpec((HID, TILE), lambda s: (0, jnp.minimum(s, NT - 1))),
            pl.BlockSpec((1, TILE), lambda s: (0, jnp.minimum(s, NT - 1))),
        ],
        out_specs=pl.BlockSpec((1, TILE), lambda s: (0, jnp.maximum(s - NT, 0))),
        out_shape=jax.ShapeDtypeStruct((1, VOCAB), jnp.float32),
        scratch_shapes=[
            pltpu.VMEM((1, HID), jnp.float32),
            pltpu.VMEM((1, NT * TILE), jnp.float32),
            pltpu.SMEM((1, 1), jnp.float32),
        ],
    )(e, W1, b1.reshape(1, HID), w2t, b2.reshape(1, VOCAB))
    return out
